# Initial kernel scaffold; baseline (speedup 1.0000x reference)
import jax
import jax.numpy as jnp
from jax.experimental import pallas as pl


def kernel(x):
    raise NotImplementedError("write your pallas kernel here")


# speedup vs baseline: 1.5960x; 1.5960x over previous
import functools

import jax
import jax.numpy as jnp
from jax.experimental import pallas as pl
from jax.experimental.pallas import tpu as pltpu

_BR = 256
_BC = 512


def _cumsum_kernel(x_ref, o_ref, carry_ref, *, bc):
    j = pl.program_id(1)

    @pl.when(j == 0)
    def _():
        carry_ref[...] = jnp.zeros_like(carry_ref)

    blk = x_ref[...]
    col = jax.lax.broadcasted_iota(jnp.int32, blk.shape, 1)
    d = 1
    while d < bc:
        rolled = pltpu.roll(blk, d, 1)
        blk = blk + jnp.where(col >= d, rolled, 0.0)
        d *= 2

    carry = carry_ref[...]
    o_ref[...] = blk + carry[:, :1]
    carry_ref[...] = carry + blk[:, bc - 1 : bc]


@jax.jit
def kernel(x):
    m, n = x.shape
    grid = (m // _BR, n // _BC)
    return pl.pallas_call(
        functools.partial(_cumsum_kernel, bc=_BC),
        grid=grid,
        in_specs=[pl.BlockSpec((_BR, _BC), lambda i, j: (i, j))],
        out_specs=pl.BlockSpec((_BR, _BC), lambda i, j: (i, j)),
        out_shape=jax.ShapeDtypeStruct((m, n), x.dtype),
        scratch_shapes=[pltpu.VMEM((_BR, 128), jnp.float32)],
        compiler_params=pltpu.CompilerParams(
            dimension_semantics=("parallel", "arbitrary")
        ),
    )(x)


# speedup vs baseline: 1.9053x; 1.1938x over previous
import functools

import jax
import jax.numpy as jnp
from jax.experimental import pallas as pl
from jax.experimental.pallas import tpu as pltpu

_BR = 256
_L = 128


def _cumsum_kernel(t_ref, x_ref, o_ref, *, br, g, l):
    xb = x_ref[...]
    x2 = xb.reshape(br * g, l)
    s2 = jnp.dot(x2, t_ref[...], preferred_element_type=jnp.float32)
    s3 = s2.reshape(br, g, l)

    tot = jnp.sum(xb, axis=2, keepdims=True)
    g_idx = jax.lax.broadcasted_iota(jnp.int32, (br, g, 1), 1)
    acc = tot
    d = 1
    while d < g:
        rolled = pltpu.roll(acc, d, 1)
        acc = acc + jnp.where(g_idx >= d, rolled, 0.0)
        d *= 2
    excl = acc - tot

    o_ref[...] = s3 + excl


@jax.jit
def kernel(x):
    m, n = x.shape
    g = n // _L
    xr = x.reshape(m, g, _L)
    tri = jnp.triu(jnp.ones((_L, _L), dtype=jnp.float32))
    out = pl.pallas_call(
        functools.partial(_cumsum_kernel, br=_BR, g=g, l=_L),
        grid=(m // _BR,),
        in_specs=[
            pl.BlockSpec((_L, _L), lambda i: (0, 0)),
            pl.BlockSpec((_BR, g, _L), lambda i: (i, 0, 0)),
        ],
        out_specs=pl.BlockSpec((_BR, g, _L), lambda i: (i, 0, 0)),
        out_shape=jax.ShapeDtypeStruct((m, g, _L), x.dtype),
        compiler_params=pltpu.CompilerParams(
            dimension_semantics=("arbitrary",)
        ),
    )(tri, xr)
    return out.reshape(m, n)
